# baseline (speedup 1.0000x reference)
import functools

import jax
import jax.numpy as jnp
from jax import lax
from jax.experimental import pallas as pl
from jax.experimental.pallas import tpu as pltpu
from jax.experimental.pallas import tpu_sc as plsc

NC = 2
NS = 16
NW = NC * NS
CH = 128


def _mesh():
    return plsc.VectorSubcoreMesh(core_axis_name="c", subcore_axis_name="s")


def _rsqrt_nr(x):
    i = plsc.bitcast(x, jnp.int32)
    i = jnp.int32(0x5F3759DF) - lax.shift_right_arithmetic(i, 1)
    y = plsc.bitcast(i, jnp.float32)
    for _ in range(3):
        y = y * (1.5 - 0.5 * x * y * y)
    return y


def _make_dinv(npad, nblk, chunk):
    ew = 2 * nblk * chunk

    @functools.partial(
        pl.kernel,
        mesh=_mesh(),
        out_type=jax.ShapeDtypeStruct((npad,), jnp.float32),
        scratch_types=[
            pltpu.VMEM_SHARED((npad,), jnp.float32),
            pltpu.VMEM((npad,), jnp.float32),
            pltpu.VMEM((npad // chunk, chunk), jnp.int32),
            pltpu.VMEM((2 * nblk, chunk), jnp.int32),
            pltpu.VMEM((npad // NS,), jnp.float32),
        ],
        compiler_params=pltpu.CompilerParams(
            use_tc_tiling_on_sc=False, needs_layout_passes=False),
    )
    def dinv_kernel(dst_hbm, out_hbm, acc_sh, deg_v, iota_v, didx, red_v):
        c = lax.axis_index("c")
        s = lax.axis_index("s")
        stripe = npad // NS
        sl = pl.ds(s * stripe, stripe)
        base16 = lax.iota(jnp.int32, 16)

        per_row = chunk // 16

        def zero(i, carry):
            dsl = pl.ds(i * 16, 16)
            deg_v[dsl] = jnp.zeros((16,), jnp.float32)
            iota_v[i // per_row, pl.ds((i % per_row) * 16, 16)] = base16 + i * 16
            return carry

        lax.fori_loop(0, npad // 16, zero, 0)
        pltpu.sync_copy(deg_v.at[sl], acc_sh.at[sl])
        pltpu.sync_copy(dst_hbm.at[2 * s], didx.at[pl.ds(0, nblk)])
        pltpu.sync_copy(dst_hbm.at[2 * s + 1], didx.at[pl.ds(nblk, nblk)])
        ones = jnp.ones((16,), jnp.float32)
        plsc.subcore_barrier()

        def body(i, carry):
            r = i // per_row
            k = (i % per_row) * 16
            idx = didx[r, pl.ds(k, 16)]
            plsc.addupdate_scatter(deg_v, [idx], ones)
            return carry

        lax.fori_loop(0, ew // 16, body, 0)
        for q in range(npad // chunk):
            pltpu.sync_copy(deg_v.at[pl.ds(q * chunk, chunk)],
                            acc_sh.at[iota_v.at[q]], add=True)
        plsc.subcore_barrier()
        pltpu.sync_copy(acc_sh.at[sl], red_v)

        def red(i, carry):
            dsl = pl.ds(i * 16, 16)
            red_v[dsl] = _rsqrt_nr(red_v[dsl] + 1.0)
            return carry

        lax.fori_loop(0, stripe // 16, red, 0)
        half = stripe // 2
        hsl = pl.ds(c * half, half)
        osl = pl.ds(s * stripe + c * half, half)
        pltpu.sync_copy(red_v.at[hsl], out_hbm.at[osl])

    return dinv_kernel


def _make_prop(npad, nch, f, nbuf, kblk, stage):
    assert nch % (nbuf * kblk) == 0
    ngrp = nch // (nbuf * kblk)

    @functools.partial(
        pl.kernel,
        mesh=_mesh(),
        out_type=jax.ShapeDtypeStruct((NC, npad, f), jnp.float32),
        scratch_types=[
            pltpu.VMEM_SHARED((npad, f), jnp.float32),
            pltpu.VMEM_SHARED((npad if stage else 8, f), jnp.float32),
            pltpu.VMEM((nch // kblk, kblk * CH), jnp.int32),
            pltpu.VMEM((nch // kblk, kblk * CH), jnp.int32),
            pltpu.VMEM((nbuf, kblk * CH, f), jnp.float32),
            pltpu.VMEM((npad // NS, f), jnp.float32),
            [pltpu.SemaphoreType.DMA] * nbuf,
            [pltpu.SemaphoreType.DMA] * nbuf,
        ],
        compiler_params=pltpu.CompilerParams(use_tc_tiling_on_sc=False),
    )
    def prop_kernel(h_hbm, src_hbm, dst_hbm, out_hbm,
                    acc_sh, h_sh, sidx, didx, rows, bounce, gsem, ssem):
        c = lax.axis_index("c")
        s = lax.axis_index("s")
        wid = s * NC + c
        stripe = npad // NS
        sl = pl.ds(s * stripe, stripe)
        pltpu.sync_copy(h_hbm.at[sl], bounce)
        pltpu.sync_copy(bounce, acc_sh.at[sl])
        if stage:
            pltpu.sync_copy(bounce, h_sh.at[sl])
        pltpu.sync_copy(src_hbm.at[wid], sidx)
        pltpu.sync_copy(dst_hbm.at[wid], didx)
        plsc.subcore_barrier()
        gsrc = h_sh if stage else h_hbm

        def gather(q, b):
            pltpu.async_copy(gsrc.at[sidx.at[q]], rows.at[b], gsem[b])

        def gather_wait(q, b):
            pltpu.make_async_copy(gsrc.at[sidx.at[q]], rows.at[b], gsem[b]).wait()

        def scatter(q, b):
            pltpu.async_copy(rows.at[b], acc_sh.at[didx.at[q]], ssem[b], add=True)

        def scatter_wait(q, b):
            pltpu.make_async_copy(rows.at[b], acc_sh.at[didx.at[q]], ssem[b]).wait()

        for b in range(nbuf):
            gather(b, b)

        def body(g, carry):
            for b in range(nbuf):
                q = g * nbuf + b
                gather_wait(q, b)
                scatter(q, b)
            for b in range(nbuf):
                q = g * nbuf + b
                scatter_wait(q, b)

                @pl.when(g + 1 < ngrp)
                def _():
                    gather(q + nbuf, b)

            return carry

        lax.fori_loop(0, ngrp, body, 0)
        plsc.subcore_barrier()
        pltpu.sync_copy(acc_sh.at[sl], bounce)
        pltpu.sync_copy(bounce, out_hbm.at[c, sl])

    return prop_kernel


def _make_prop_cols(npad, nblk, fh, nbuf, chunk):
    nq = 2 * nblk
    assert nq % nbuf == 0
    ngrp = nq // nbuf

    @functools.partial(
        pl.kernel,
        mesh=_mesh(),
        out_type=jax.ShapeDtypeStruct((NC, npad, fh), jnp.float32),
        scratch_types=[
            pltpu.VMEM_SHARED((npad, fh), jnp.float32),
            pltpu.VMEM_SHARED((npad, fh), jnp.float32),
            pltpu.VMEM((nq, chunk), jnp.int32),
            pltpu.VMEM((nq, chunk), jnp.int32),
            pltpu.VMEM((nbuf, chunk, fh), jnp.float32),
            pltpu.VMEM((npad // NS, fh), jnp.float32),
            [pltpu.SemaphoreType.DMA] * nbuf,
            [pltpu.SemaphoreType.DMA] * nbuf,
        ],
        compiler_params=pltpu.CompilerParams(use_tc_tiling_on_sc=False),
    )
    def prop_kernel(hl_hbm, hr_hbm, src_hbm, dst_hbm, out_hbm,
                    acc_sh, h_sh, sidx, didx, rows, bounce, gsem, ssem):
        c = lax.axis_index("c")
        s = lax.axis_index("s")
        stripe = npad // NS
        sl = pl.ds(s * stripe, stripe)

        @pl.when(c == 0)
        def _():
            pltpu.sync_copy(hl_hbm.at[sl], bounce)

        @pl.when(c == 1)
        def _():
            pltpu.sync_copy(hr_hbm.at[sl], bounce)

        pltpu.sync_copy(bounce, acc_sh.at[sl])
        pltpu.sync_copy(bounce, h_sh.at[sl])
        pltpu.sync_copy(src_hbm.at[2 * s], sidx.at[pl.ds(0, nblk)])
        pltpu.sync_copy(src_hbm.at[2 * s + 1], sidx.at[pl.ds(nblk, nblk)])
        pltpu.sync_copy(dst_hbm.at[2 * s], didx.at[pl.ds(0, nblk)])
        pltpu.sync_copy(dst_hbm.at[2 * s + 1], didx.at[pl.ds(nblk, nblk)])
        plsc.subcore_barrier()

        def gather(q, b):
            pltpu.async_copy(h_sh.at[sidx.at[q]], rows.at[b], gsem[b])

        def gather_wait(q, b):
            pltpu.make_async_copy(h_sh.at[sidx.at[q]], rows.at[b], gsem[b]).wait()

        def scatter(q, b):
            pltpu.async_copy(rows.at[b], acc_sh.at[didx.at[q]], ssem[b], add=True)

        def scatter_wait(q, b):
            pltpu.make_async_copy(rows.at[b], acc_sh.at[didx.at[q]], ssem[b]).wait()

        for b in range(nbuf):
            gather(b, b)

        def body(g, carry):
            for b in range(nbuf):
                q = g * nbuf + b
                gather_wait(q, b)
                scatter(q, b)
            for b in range(nbuf):
                q = g * nbuf + b
                scatter_wait(q, b)

                @pl.when(g + 1 < ngrp)
                def _():
                    gather(q + nbuf, b)

            return carry

        lax.fori_loop(0, ngrp, body, 0)
        plsc.subcore_barrier()
        pltpu.sync_copy(acc_sh.at[sl], bounce)
        pltpu.sync_copy(bounce, out_hbm.at[c, sl])

    return prop_kernel


def _tc_first(x, w1, dinv):
    npad = x.shape[0]
    h1dim = w1.shape[1]
    fh = h1dim // 2

    def body(x_ref, w_ref, dinv_ref, hl_ref, hr_ref):
        h = jnp.dot(x_ref[...], w_ref[...], preferred_element_type=jnp.float32)
        hs = h * dinv_ref[...]
        hl_ref[...] = hs[:, :fh]
        hr_ref[...] = hs[:, fh:]

    return pl.pallas_call(
        body,
        out_shape=(
            jax.ShapeDtypeStruct((npad, fh), jnp.float32),
            jax.ShapeDtypeStruct((npad, fh), jnp.float32),
        ),
    )(x, w1, dinv)


def _tc_mid(a0, a1, dinv, w2p):
    npad = a0.shape[0]
    f2 = w2p.shape[1]

    def body(a0_ref, a1_ref, dinv_ref, w_ref, out_ref):
        t = jnp.concatenate([a0_ref[...], a1_ref[...]], axis=1) * dinv_ref[...]
        g = jnp.maximum(t, 0.0)
        h2 = jnp.dot(g, w_ref[...], preferred_element_type=jnp.float32)
        out_ref[...] = h2 * dinv_ref[...]

    return pl.pallas_call(
        body,
        out_shape=jax.ShapeDtypeStruct((npad, f2), jnp.float32),
    )(a0, a1, dinv, w2p)


def _tc_final(a0, a1, h2s, dinv):
    npad, f2 = a0.shape

    def body(a0_ref, a1_ref, h_ref, dinv_ref, out_ref):
        out_ref[...] = (a0_ref[...] + a1_ref[...] - h_ref[...]) * dinv_ref[...]

    return pl.pallas_call(
        body,
        out_shape=jax.ShapeDtypeStruct((npad, f2), jnp.float32),
    )(a0, a1, h2s, dinv)


def kernel(node_features, edge_features, latent_features, edge_index, device, W1, W2):
    x = node_features
    n = x.shape[0]
    e = edge_index.shape[1]
    h1dim = W1.shape[1]
    h2dim = W2.shape[1]
    f2 = 16

    kblk = 8
    chunk = kblk * CH
    nacc = ((n + 1 + 127) // 128) * 128
    ndeg = ((n + 1 + 255) // 256) * 256
    nblk = (e + NW * chunk - 1) // (NW * chunk)
    nblk = ((nblk + 1) // 2) * 2
    nch = nblk * kblk
    etot = NW * nblk * chunk

    src = edge_index[0]
    dst = edge_index[1]
    src_b = jnp.concatenate(
        [src, jnp.zeros((etot - e,), jnp.int32)]).reshape(NW, nblk, chunk)
    dst_b = jnp.concatenate(
        [dst, jnp.full((etot - e,), n, jnp.int32)]).reshape(NW, nblk, chunk)
    x_p = jnp.pad(x, ((0, nacc - n), (0, 0)))
    w2p = jnp.pad(W2, ((0, 0), (0, f2 - h2dim)))

    dinv = _make_dinv(ndeg, nblk, chunk)(dst_b)
    dinv2d = dinv[:nacc].reshape(nacc, 1)
    hl, hr = _tc_first(x_p, W1, dinv2d)
    acc1 = _make_prop_cols(nacc, nblk, h1dim // 2, 2, chunk)(hl, hr, src_b, dst_b)
    h2s = _tc_mid(acc1[0], acc1[1], dinv2d, w2p)
    acc2 = _make_prop(nacc, nch, f2, 2, kblk, False)(h2s, src_b, dst_b)
    outp = _tc_final(acc2[0], acc2[1], h2s, dinv2d)
    return outp[:n, :h2dim]


# speedup vs baseline: 1.1236x; 1.1236x over previous
import functools

import jax
import jax.numpy as jnp
from jax import lax
from jax.experimental import pallas as pl
from jax.experimental.pallas import tpu as pltpu
from jax.experimental.pallas import tpu_sc as plsc

NC = 2
NS = 16
NW = NC * NS
CH = 128


def _mesh():
    return plsc.VectorSubcoreMesh(core_axis_name="c", subcore_axis_name="s")


def _rsqrt_nr(x):
    i = plsc.bitcast(x, jnp.int32)
    i = jnp.int32(0x5F3759DF) - lax.shift_right_arithmetic(i, 1)
    y = plsc.bitcast(i, jnp.float32)
    for _ in range(3):
        y = y * (1.5 - 0.5 * x * y * y)
    return y


def _make_dinv(npad, nblk, chunk):
    ew = 2 * nblk * chunk

    @functools.partial(
        pl.kernel,
        mesh=_mesh(),
        out_type=jax.ShapeDtypeStruct((npad,), jnp.float32),
        scratch_types=[
            pltpu.VMEM_SHARED((npad,), jnp.float32),
            pltpu.VMEM((npad,), jnp.float32),
            pltpu.VMEM((npad // chunk, chunk), jnp.int32),
            pltpu.VMEM((2 * nblk, chunk), jnp.int32),
            pltpu.VMEM((npad // NS,), jnp.float32),
        ],
        compiler_params=pltpu.CompilerParams(
            use_tc_tiling_on_sc=False, needs_layout_passes=False),
    )
    def dinv_kernel(dst_hbm, out_hbm, acc_sh, deg_v, iota_v, didx, red_v):
        c = lax.axis_index("c")
        s = lax.axis_index("s")
        stripe = npad // NS
        sl = pl.ds(s * stripe, stripe)
        base16 = lax.iota(jnp.int32, 16)

        per_row = chunk // 16

        def zero(i, carry):
            dsl = pl.ds(i * 16, 16)
            deg_v[dsl] = jnp.zeros((16,), jnp.float32)
            iota_v[i // per_row, pl.ds((i % per_row) * 16, 16)] = base16 + i * 16
            return carry

        lax.fori_loop(0, npad // 16, zero, 0)
        pltpu.sync_copy(deg_v.at[sl], acc_sh.at[sl])
        pltpu.sync_copy(dst_hbm.at[2 * s], didx.at[pl.ds(0, nblk)])
        pltpu.sync_copy(dst_hbm.at[2 * s + 1], didx.at[pl.ds(nblk, nblk)])
        ones = jnp.ones((16,), jnp.float32)
        plsc.subcore_barrier()

        def body(i, carry):
            r = i // per_row
            k = (i % per_row) * 16
            idx = didx[r, pl.ds(k, 16)]
            plsc.addupdate_scatter(deg_v, [idx], ones)
            return carry

        lax.fori_loop(0, ew // 16, body, 0)
        for q in range(npad // chunk):
            pltpu.sync_copy(deg_v.at[pl.ds(q * chunk, chunk)],
                            acc_sh.at[iota_v.at[q]], add=True)
        plsc.subcore_barrier()
        pltpu.sync_copy(acc_sh.at[sl], red_v)

        def red(i, carry):
            dsl = pl.ds(i * 16, 16)
            red_v[dsl] = _rsqrt_nr(red_v[dsl] + 1.0)
            return carry

        lax.fori_loop(0, stripe // 16, red, 0)
        half = stripe // 2
        hsl = pl.ds(c * half, half)
        osl = pl.ds(s * stripe + c * half, half)
        pltpu.sync_copy(red_v.at[hsl], out_hbm.at[osl])

    return dinv_kernel


def _make_prop(npad, nch, f, nbuf, kblk, stage):
    assert nch % (nbuf * kblk) == 0
    ngrp = nch // (nbuf * kblk)

    @functools.partial(
        pl.kernel,
        mesh=_mesh(),
        out_type=jax.ShapeDtypeStruct((NC, npad, f), jnp.float32),
        scratch_types=[
            pltpu.VMEM_SHARED((npad, f), jnp.float32),
            pltpu.VMEM_SHARED((npad if stage else 8, f), jnp.float32),
            pltpu.VMEM((nch // kblk, kblk * CH), jnp.int32),
            pltpu.VMEM((nch // kblk, kblk * CH), jnp.int32),
            pltpu.VMEM((nbuf, kblk * CH, f), jnp.float32),
            pltpu.VMEM((npad // NS, f), jnp.float32),
            [pltpu.SemaphoreType.DMA] * nbuf,
            [pltpu.SemaphoreType.DMA] * nbuf,
        ],
        compiler_params=pltpu.CompilerParams(use_tc_tiling_on_sc=False),
    )
    def prop_kernel(h_hbm, src_hbm, dst_hbm, out_hbm,
                    acc_sh, h_sh, sidx, didx, rows, bounce, gsem, ssem):
        c = lax.axis_index("c")
        s = lax.axis_index("s")
        wid = s * NC + c
        stripe = npad // NS
        sl = pl.ds(s * stripe, stripe)
        pltpu.sync_copy(h_hbm.at[sl], bounce)
        pltpu.sync_copy(bounce, acc_sh.at[sl])
        if stage:
            pltpu.sync_copy(bounce, h_sh.at[sl])
        pltpu.sync_copy(src_hbm.at[wid], sidx)
        pltpu.sync_copy(dst_hbm.at[wid], didx)
        plsc.subcore_barrier()
        gsrc = h_sh if stage else h_hbm

        def gather(q, b):
            pltpu.async_copy(gsrc.at[sidx.at[q]], rows.at[b], gsem[b])

        def gather_wait(q, b):
            pltpu.make_async_copy(gsrc.at[sidx.at[q]], rows.at[b], gsem[b]).wait()

        def scatter(q, b):
            pltpu.async_copy(rows.at[b], acc_sh.at[didx.at[q]], ssem[b], add=True)

        def scatter_wait(q, b):
            pltpu.make_async_copy(rows.at[b], acc_sh.at[didx.at[q]], ssem[b]).wait()

        for b in range(nbuf):
            gather(b, b)

        def body(g, carry):
            for b in range(nbuf):
                q = g * nbuf + b
                gather_wait(q, b)
                scatter(q, b)
            for b in range(nbuf):
                q = g * nbuf + b
                scatter_wait(q, b)

                @pl.when(g + 1 < ngrp)
                def _():
                    gather(q + nbuf, b)

            return carry

        lax.fori_loop(0, ngrp, body, 0)
        plsc.subcore_barrier()
        pltpu.sync_copy(acc_sh.at[sl], bounce)
        pltpu.sync_copy(bounce, out_hbm.at[c, sl])

    return prop_kernel


def _make_prop_cols(npad, nblk, fh, nbuf, chunk):
    nq = 2 * nblk
    assert nq % nbuf == 0
    ngrp = nq // nbuf

    @functools.partial(
        pl.kernel,
        mesh=_mesh(),
        out_type=jax.ShapeDtypeStruct((NC, npad, fh), jnp.float32),
        scratch_types=[
            pltpu.VMEM_SHARED((npad, fh), jnp.float32),
            pltpu.VMEM_SHARED((npad, fh), jnp.float32),
            pltpu.VMEM((nq, chunk), jnp.int32),
            pltpu.VMEM((nq, chunk), jnp.int32),
            pltpu.VMEM((nbuf, chunk, fh), jnp.float32),
            pltpu.VMEM((npad // NS, fh), jnp.float32),
            [pltpu.SemaphoreType.DMA] * nbuf,
            [pltpu.SemaphoreType.DMA] * nbuf,
        ],
        compiler_params=pltpu.CompilerParams(use_tc_tiling_on_sc=False),
    )
    def prop_kernel(hl_hbm, hr_hbm, src_hbm, dst_hbm, out_hbm,
                    acc_sh, h_sh, sidx, didx, rows, bounce, gsem, ssem):
        c = lax.axis_index("c")
        s = lax.axis_index("s")
        stripe = npad // NS
        sl = pl.ds(s * stripe, stripe)

        @pl.when(c == 0)
        def _():
            pltpu.sync_copy(hl_hbm.at[sl], bounce)

        @pl.when(c == 1)
        def _():
            pltpu.sync_copy(hr_hbm.at[sl], bounce)

        pltpu.sync_copy(bounce, acc_sh.at[sl])
        pltpu.sync_copy(bounce, h_sh.at[sl])
        pltpu.sync_copy(src_hbm.at[2 * s], sidx.at[pl.ds(0, nblk)])
        pltpu.sync_copy(src_hbm.at[2 * s + 1], sidx.at[pl.ds(nblk, nblk)])
        pltpu.sync_copy(dst_hbm.at[2 * s], didx.at[pl.ds(0, nblk)])
        pltpu.sync_copy(dst_hbm.at[2 * s + 1], didx.at[pl.ds(nblk, nblk)])
        plsc.subcore_barrier()

        def gather(q, b):
            pltpu.async_copy(h_sh.at[sidx.at[q]], rows.at[b], gsem[b])

        def gather_wait(q, b):
            pltpu.make_async_copy(h_sh.at[sidx.at[q]], rows.at[b], gsem[b]).wait()

        def scatter(q, b):
            pltpu.async_copy(rows.at[b], acc_sh.at[didx.at[q]], ssem[b], add=True)

        def scatter_wait(q, b):
            pltpu.make_async_copy(rows.at[b], acc_sh.at[didx.at[q]], ssem[b]).wait()

        for b in range(nbuf):
            gather(b, b)

        def body(g, carry):
            for b in range(nbuf):
                q = g * nbuf + b
                gather_wait(q, b)
                scatter(q, b)
            for b in range(nbuf):
                q = g * nbuf + b
                scatter_wait(q, b)

                @pl.when(g + 1 < ngrp)
                def _():
                    gather(q + nbuf, b)

            return carry

        lax.fori_loop(0, ngrp, body, 0)
        plsc.subcore_barrier()
        pltpu.sync_copy(acc_sh.at[sl], bounce)
        pltpu.sync_copy(bounce, out_hbm.at[c, sl])

    return prop_kernel


def _tc_first(x, w1, dinv):
    npad = x.shape[0]
    h1dim = w1.shape[1]
    fh = h1dim // 2

    def body(x_ref, w_ref, dinv_ref, hl_ref, hr_ref):
        h = jnp.dot(x_ref[...], w_ref[...], preferred_element_type=jnp.float32)
        hs = h * dinv_ref[...]
        hl_ref[...] = hs[:, :fh]
        hr_ref[...] = hs[:, fh:]

    return pl.pallas_call(
        body,
        out_shape=(
            jax.ShapeDtypeStruct((npad, fh), jnp.float32),
            jax.ShapeDtypeStruct((npad, fh), jnp.float32),
        ),
    )(x, w1, dinv)


def _tc_mid(a0, a1, dinv, w2p):
    npad = a0.shape[0]
    f2 = w2p.shape[1]

    def body(a0_ref, a1_ref, dinv_ref, w_ref, out_ref):
        t = jnp.concatenate([a0_ref[...], a1_ref[...]], axis=1) * dinv_ref[...]
        g = jnp.maximum(t, 0.0)
        h2 = jnp.dot(g, w_ref[...], preferred_element_type=jnp.float32)
        out_ref[...] = h2 * dinv_ref[...]

    return pl.pallas_call(
        body,
        out_shape=jax.ShapeDtypeStruct((npad, f2), jnp.float32),
    )(a0, a1, dinv, w2p)


def _tc_final(a0, a1, h2s, dinv):
    npad, f2 = a0.shape

    def body(a0_ref, a1_ref, h_ref, dinv_ref, out_ref):
        out_ref[...] = (a0_ref[...] + a1_ref[...] - h_ref[...]) * dinv_ref[...]

    return pl.pallas_call(
        body,
        out_shape=jax.ShapeDtypeStruct((npad, f2), jnp.float32),
    )(a0, a1, h2s, dinv)


def kernel(node_features, edge_features, latent_features, edge_index, device, W1, W2):
    x = node_features
    n = x.shape[0]
    e = edge_index.shape[1]
    h1dim = W1.shape[1]
    h2dim = W2.shape[1]
    f2 = 16

    kblk = 8
    chunk = kblk * CH
    nacc = ((n + 1 + 127) // 128) * 128
    ndeg = ((n + 1 + 255) // 256) * 256
    nblk = (e + NW * chunk - 1) // (NW * chunk)
    nblk = ((nblk + 1) // 2) * 2
    nch = nblk * kblk
    etot = NW * nblk * chunk

    src = edge_index[0]
    dst = edge_index[1]
    src_b = jnp.concatenate(
        [src, jnp.zeros((etot - e,), jnp.int32)]).reshape(NW, nblk, chunk)
    dst_b = jnp.concatenate(
        [dst, jnp.full((etot - e,), n, jnp.int32)]).reshape(NW, nblk, chunk)
    x_p = jnp.pad(x, ((0, nacc - n), (0, 0)))
    w2p = jnp.pad(W2, ((0, 0), (0, f2 - h2dim)))

    dinv = _make_dinv(ndeg, nblk, chunk)(dst_b)
    dinv2d = dinv[:nacc].reshape(nacc, 1)
    hl, hr = _tc_first(x_p, W1, dinv2d)
    acc1 = _make_prop_cols(nacc, nblk, h1dim // 2, 2, chunk)(hl, hr, src_b, dst_b)
    h2s = _tc_mid(acc1[0], acc1[1], dinv2d, w2p)
    acc2 = _make_prop(nacc, nch, f2, 2, kblk, True)(h2s, src_b, dst_b)
    outp = _tc_final(acc2[0], acc2[1], h2s, dinv2d)
    return outp[:n, :h2dim]


# speedup vs baseline: 1.1326x; 1.0080x over previous
import functools

import jax
import jax.numpy as jnp
from jax import lax
from jax.experimental import pallas as pl
from jax.experimental.pallas import tpu as pltpu
from jax.experimental.pallas import tpu_sc as plsc

NC = 2
NS = 16
NW = NC * NS
CH = 128


def _mesh():
    return plsc.VectorSubcoreMesh(core_axis_name="c", subcore_axis_name="s")


def _rsqrt_nr(x):
    i = plsc.bitcast(x, jnp.int32)
    i = jnp.int32(0x5F3759DF) - lax.shift_right_arithmetic(i, 1)
    y = plsc.bitcast(i, jnp.float32)
    for _ in range(3):
        y = y * (1.5 - 0.5 * x * y * y)
    return y


def _make_dinv(npad, nblk, chunk):
    ew = 2 * nblk * chunk

    nrow = npad // 16

    @functools.partial(
        pl.kernel,
        mesh=_mesh(),
        out_type=jax.ShapeDtypeStruct((nrow, 16), jnp.float32),
        scratch_types=[
            pltpu.VMEM_SHARED((nrow, 16), jnp.float32),
            pltpu.VMEM((nrow, 16), jnp.float32),
            pltpu.VMEM((nrow,), jnp.int32),
            pltpu.VMEM((2 * nblk, chunk), jnp.int32),
            pltpu.VMEM((nrow // NS, 16), jnp.float32),
        ],
        compiler_params=pltpu.CompilerParams(
            use_tc_tiling_on_sc=False, needs_layout_passes=False),
    )
    def dinv_kernel(dst_hbm, out_hbm, acc_sh, deg_v, iota_v, didx, red_v):
        c = lax.axis_index("c")
        s = lax.axis_index("s")
        rstripe = nrow // NS
        rsl = pl.ds(s * rstripe, rstripe)
        base16 = lax.iota(jnp.int32, 16)
        per_row = chunk // 16

        def zero(i, carry):
            deg_v[i, pl.ds(0, 16)] = jnp.zeros((16,), jnp.float32)
            return carry

        lax.fori_loop(0, nrow, zero, 0)

        def fill_iota(j, carry):
            iota_v[pl.ds(j * 16, 16)] = base16 + j * 16
            return carry

        lax.fori_loop(0, nrow // 16, fill_iota, 0)
        pltpu.sync_copy(deg_v.at[rsl], acc_sh.at[rsl])
        pltpu.sync_copy(dst_hbm.at[2 * s], didx.at[pl.ds(0, nblk)])
        pltpu.sync_copy(dst_hbm.at[2 * s + 1], didx.at[pl.ds(nblk, nblk)])
        ones = jnp.ones((16,), jnp.float32)
        plsc.subcore_barrier()

        def body(i, carry):
            r = i // per_row
            k = (i % per_row) * 16
            idx = didx[r, pl.ds(k, 16)]
            plsc.addupdate_scatter(
                deg_v,
                [lax.shift_right_logical(idx, 4),
                 lax.bitwise_and(idx, jnp.int32(15))],
                ones)
            return carry

        lax.fori_loop(0, ew // 16, body, 0)
        pltpu.sync_copy(deg_v, acc_sh.at[iota_v], add=True)
        plsc.subcore_barrier()
        pltpu.sync_copy(acc_sh.at[rsl], red_v)

        def red(i, carry):
            red_v[i, pl.ds(0, 16)] = _rsqrt_nr(red_v[i, pl.ds(0, 16)] + 1.0)
            return carry

        lax.fori_loop(0, rstripe, red, 0)
        half = rstripe // 2
        hsl = pl.ds(c * half, half)
        osl = pl.ds(s * rstripe + c * half, half)
        pltpu.sync_copy(red_v.at[hsl], out_hbm.at[osl])

    return dinv_kernel


def _make_prop(npad, nch, f, nbuf, kblk, stage):
    assert nch % (nbuf * kblk) == 0
    ngrp = nch // (nbuf * kblk)

    @functools.partial(
        pl.kernel,
        mesh=_mesh(),
        out_type=jax.ShapeDtypeStruct((NC, npad, f), jnp.float32),
        scratch_types=[
            pltpu.VMEM_SHARED((npad, f), jnp.float32),
            pltpu.VMEM_SHARED((npad if stage else 8, f), jnp.float32),
            pltpu.VMEM((nch // kblk, kblk * CH), jnp.int32),
            pltpu.VMEM((nch // kblk, kblk * CH), jnp.int32),
            pltpu.VMEM((nbuf, kblk * CH, f), jnp.float32),
            pltpu.VMEM((npad // NS, f), jnp.float32),
            [pltpu.SemaphoreType.DMA] * nbuf,
            [pltpu.SemaphoreType.DMA] * nbuf,
        ],
        compiler_params=pltpu.CompilerParams(use_tc_tiling_on_sc=False),
    )
    def prop_kernel(h_hbm, src_hbm, dst_hbm, out_hbm,
                    acc_sh, h_sh, sidx, didx, rows, bounce, gsem, ssem):
        c = lax.axis_index("c")
        s = lax.axis_index("s")
        wid = s * NC + c
        stripe = npad // NS
        sl = pl.ds(s * stripe, stripe)
        pltpu.sync_copy(h_hbm.at[sl], bounce)
        pltpu.sync_copy(bounce, acc_sh.at[sl])
        if stage:
            pltpu.sync_copy(bounce, h_sh.at[sl])
        pltpu.sync_copy(src_hbm.at[wid], sidx)
        pltpu.sync_copy(dst_hbm.at[wid], didx)
        plsc.subcore_barrier()
        gsrc = h_sh if stage else h_hbm

        def gather(q, b):
            pltpu.async_copy(gsrc.at[sidx.at[q]], rows.at[b], gsem[b])

        def gather_wait(q, b):
            pltpu.make_async_copy(gsrc.at[sidx.at[q]], rows.at[b], gsem[b]).wait()

        def scatter(q, b):
            pltpu.async_copy(rows.at[b], acc_sh.at[didx.at[q]], ssem[b], add=True)

        def scatter_wait(q, b):
            pltpu.make_async_copy(rows.at[b], acc_sh.at[didx.at[q]], ssem[b]).wait()

        for b in range(nbuf):
            gather(b, b)

        def body(g, carry):
            for b in range(nbuf):
                q = g * nbuf + b
                gather_wait(q, b)
                scatter(q, b)
            for b in range(nbuf):
                q = g * nbuf + b
                scatter_wait(q, b)

                @pl.when(g + 1 < ngrp)
                def _():
                    gather(q + nbuf, b)

            return carry

        lax.fori_loop(0, ngrp, body, 0)
        plsc.subcore_barrier()
        pltpu.sync_copy(acc_sh.at[sl], bounce)
        pltpu.sync_copy(bounce, out_hbm.at[c, sl])

    return prop_kernel


def _make_prop_cols(npad, nblk, fh, nbuf, chunk):
    nq = 2 * nblk
    assert nq % nbuf == 0
    ngrp = nq // nbuf

    @functools.partial(
        pl.kernel,
        mesh=_mesh(),
        out_type=jax.ShapeDtypeStruct((NC, npad, fh), jnp.float32),
        scratch_types=[
            pltpu.VMEM_SHARED((npad, fh), jnp.float32),
            pltpu.VMEM_SHARED((npad, fh), jnp.float32),
            pltpu.VMEM((nq, chunk), jnp.int32),
            pltpu.VMEM((nq, chunk), jnp.int32),
            pltpu.VMEM((nbuf, chunk, fh), jnp.float32),
            pltpu.VMEM((npad // NS, fh), jnp.float32),
            [pltpu.SemaphoreType.DMA] * nbuf,
            [pltpu.SemaphoreType.DMA] * nbuf,
        ],
        compiler_params=pltpu.CompilerParams(use_tc_tiling_on_sc=False),
    )
    def prop_kernel(hl_hbm, hr_hbm, src_hbm, dst_hbm, out_hbm,
                    acc_sh, h_sh, sidx, didx, rows, bounce, gsem, ssem):
        c = lax.axis_index("c")
        s = lax.axis_index("s")
        stripe = npad // NS
        sl = pl.ds(s * stripe, stripe)

        @pl.when(c == 0)
        def _():
            pltpu.sync_copy(hl_hbm.at[sl], bounce)

        @pl.when(c == 1)
        def _():
            pltpu.sync_copy(hr_hbm.at[sl], bounce)

        pltpu.sync_copy(bounce, acc_sh.at[sl])
        pltpu.sync_copy(bounce, h_sh.at[sl])
        pltpu.sync_copy(src_hbm.at[2 * s], sidx.at[pl.ds(0, nblk)])
        pltpu.sync_copy(src_hbm.at[2 * s + 1], sidx.at[pl.ds(nblk, nblk)])
        pltpu.sync_copy(dst_hbm.at[2 * s], didx.at[pl.ds(0, nblk)])
        pltpu.sync_copy(dst_hbm.at[2 * s + 1], didx.at[pl.ds(nblk, nblk)])
        plsc.subcore_barrier()

        def gather(q, b):
            pltpu.async_copy(h_sh.at[sidx.at[q]], rows.at[b], gsem[b])

        def gather_wait(q, b):
            pltpu.make_async_copy(h_sh.at[sidx.at[q]], rows.at[b], gsem[b]).wait()

        def scatter(q, b):
            pltpu.async_copy(rows.at[b], acc_sh.at[didx.at[q]], ssem[b], add=True)

        def scatter_wait(q, b):
            pltpu.make_async_copy(rows.at[b], acc_sh.at[didx.at[q]], ssem[b]).wait()

        for b in range(nbuf):
            gather(b, b)

        def body(g, carry):
            for b in range(nbuf):
                q = g * nbuf + b
                gather_wait(q, b)
                scatter(q, b)
            for b in range(nbuf):
                q = g * nbuf + b
                scatter_wait(q, b)

                @pl.when(g + 1 < ngrp)
                def _():
                    gather(q + nbuf, b)

            return carry

        lax.fori_loop(0, ngrp, body, 0)
        plsc.subcore_barrier()
        pltpu.sync_copy(acc_sh.at[sl], bounce)
        pltpu.sync_copy(bounce, out_hbm.at[c, sl])

    return prop_kernel


def _tc_first(x, w1, dinv):
    npad = x.shape[0]
    h1dim = w1.shape[1]
    fh = h1dim // 2

    def body(x_ref, w_ref, dinv_ref, hl_ref, hr_ref):
        h = jnp.dot(x_ref[...], w_ref[...], preferred_element_type=jnp.float32)
        hs = h * dinv_ref[...]
        hl_ref[...] = hs[:, :fh]
        hr_ref[...] = hs[:, fh:]

    return pl.pallas_call(
        body,
        out_shape=(
            jax.ShapeDtypeStruct((npad, fh), jnp.float32),
            jax.ShapeDtypeStruct((npad, fh), jnp.float32),
        ),
    )(x, w1, dinv)


def _tc_mid(a0, a1, dinv, w2p):
    npad = a0.shape[0]
    f2 = w2p.shape[1]

    def body(a0_ref, a1_ref, dinv_ref, w_ref, out_ref):
        t = jnp.concatenate([a0_ref[...], a1_ref[...]], axis=1) * dinv_ref[...]
        g = jnp.maximum(t, 0.0)
        h2 = jnp.dot(g, w_ref[...], preferred_element_type=jnp.float32)
        out_ref[...] = h2 * dinv_ref[...]

    return pl.pallas_call(
        body,
        out_shape=jax.ShapeDtypeStruct((npad, f2), jnp.float32),
    )(a0, a1, dinv, w2p)


def _tc_final(a0, a1, h2s, dinv):
    npad, f2 = a0.shape

    def body(a0_ref, a1_ref, h_ref, dinv_ref, out_ref):
        out_ref[...] = (a0_ref[...] + a1_ref[...] - h_ref[...]) * dinv_ref[...]

    return pl.pallas_call(
        body,
        out_shape=jax.ShapeDtypeStruct((npad, f2), jnp.float32),
    )(a0, a1, h2s, dinv)


def kernel(node_features, edge_features, latent_features, edge_index, device, W1, W2):
    x = node_features
    n = x.shape[0]
    e = edge_index.shape[1]
    h1dim = W1.shape[1]
    h2dim = W2.shape[1]
    f2 = 16

    kblk = 8
    chunk = kblk * CH
    nacc = ((n + 1 + 127) // 128) * 128
    ndeg = ((n + 1 + 255) // 256) * 256
    nblk = (e + NW * chunk - 1) // (NW * chunk)
    nblk = ((nblk + 1) // 2) * 2
    nch = nblk * kblk
    etot = NW * nblk * chunk

    src = edge_index[0]
    dst = edge_index[1]
    src_b = jnp.concatenate(
        [src, jnp.zeros((etot - e,), jnp.int32)]).reshape(NW, nblk, chunk)
    dst_b = jnp.concatenate(
        [dst, jnp.full((etot - e,), n, jnp.int32)]).reshape(NW, nblk, chunk)
    x_p = jnp.pad(x, ((0, nacc - n), (0, 0)))
    w2p = jnp.pad(W2, ((0, 0), (0, f2 - h2dim)))

    dinv = _make_dinv(ndeg, nblk, chunk)(dst_b)
    dinv2d = dinv.reshape(ndeg)[:nacc].reshape(nacc, 1)
    hl, hr = _tc_first(x_p, W1, dinv2d)
    acc1 = _make_prop_cols(nacc, nblk, h1dim // 2, 2, chunk)(hl, hr, src_b, dst_b)
    h2s = _tc_mid(acc1[0], acc1[1], dinv2d, w2p)
    acc2 = _make_prop(nacc, nch, f2, 2, kblk, True)(h2s, src_b, dst_b)
    outp = _tc_final(acc2[0], acc2[1], h2s, dinv2d)
    return outp[:n, :h2dim]


# speedup vs baseline: 1.1473x; 1.0130x over previous
import functools

import jax
import jax.numpy as jnp
from jax import lax
from jax.experimental import pallas as pl
from jax.experimental.pallas import tpu as pltpu
from jax.experimental.pallas import tpu_sc as plsc

NC = 2
NS = 16
NW = NC * NS
CH = 128


def _mesh():
    return plsc.VectorSubcoreMesh(core_axis_name="c", subcore_axis_name="s")


def _rsqrt_nr(x):
    i = plsc.bitcast(x, jnp.int32)
    i = jnp.int32(0x5F3759DF) - lax.shift_right_arithmetic(i, 1)
    y = plsc.bitcast(i, jnp.float32)
    for _ in range(3):
        y = y * (1.5 - 0.5 * x * y * y)
    return y


def _make_dinv(npad, nblk, chunk):
    ew = 2 * nblk * chunk

    nrow = npad // 16

    @functools.partial(
        pl.kernel,
        mesh=_mesh(),
        out_type=jax.ShapeDtypeStruct((nrow, 16), jnp.float32),
        scratch_types=[
            pltpu.VMEM_SHARED((nrow, 16), jnp.float32),
            pltpu.VMEM((nrow, 16), jnp.float32),
            pltpu.VMEM((nrow,), jnp.int32),
            pltpu.VMEM((2 * nblk, chunk), jnp.int32),
            pltpu.VMEM((nrow // NS, 16), jnp.float32),
        ],
        compiler_params=pltpu.CompilerParams(
            use_tc_tiling_on_sc=False, needs_layout_passes=False),
    )
    def dinv_kernel(dst_hbm, out_hbm, acc_sh, deg_v, iota_v, didx, red_v):
        c = lax.axis_index("c")
        s = lax.axis_index("s")
        rstripe = nrow // NS
        rsl = pl.ds(s * rstripe, rstripe)
        base16 = lax.iota(jnp.int32, 16)
        per_row = chunk // 16

        def zero(i, carry):
            deg_v[i, pl.ds(0, 16)] = jnp.zeros((16,), jnp.float32)
            return carry

        lax.fori_loop(0, nrow, zero, 0)

        def fill_iota(j, carry):
            iota_v[pl.ds(j * 16, 16)] = base16 + j * 16
            return carry

        lax.fori_loop(0, nrow // 16, fill_iota, 0)
        pltpu.sync_copy(deg_v.at[rsl], acc_sh.at[rsl])
        pltpu.sync_copy(dst_hbm.at[2 * s], didx.at[pl.ds(0, nblk)])
        pltpu.sync_copy(dst_hbm.at[2 * s + 1], didx.at[pl.ds(nblk, nblk)])
        ones = jnp.ones((16,), jnp.float32)
        plsc.subcore_barrier()

        def body(i, carry):
            r = i // per_row
            k = (i % per_row) * 16
            idx = didx[r, pl.ds(k, 16)]
            plsc.addupdate_scatter(
                deg_v,
                [lax.shift_right_logical(idx, 4),
                 lax.bitwise_and(idx, jnp.int32(15))],
                ones)
            return carry

        lax.fori_loop(0, ew // 16, body, 0)
        pltpu.sync_copy(deg_v, acc_sh.at[iota_v], add=True)
        plsc.subcore_barrier()
        pltpu.sync_copy(acc_sh.at[rsl], red_v)

        def red(i, carry):
            red_v[i, pl.ds(0, 16)] = _rsqrt_nr(red_v[i, pl.ds(0, 16)] + 1.0)
            return carry

        lax.fori_loop(0, rstripe, red, 0)
        half = rstripe // 2
        hsl = pl.ds(c * half, half)
        osl = pl.ds(s * rstripe + c * half, half)
        pltpu.sync_copy(red_v.at[hsl], out_hbm.at[osl])

    return dinv_kernel


def _make_prop(npad, nch, f, nbuf, kblk, stage):
    assert nch % (nbuf * kblk) == 0
    ngrp = nch // (nbuf * kblk)

    @functools.partial(
        pl.kernel,
        mesh=_mesh(),
        out_type=jax.ShapeDtypeStruct((NC, npad, f), jnp.float32),
        scratch_types=[
            pltpu.VMEM_SHARED((npad, f), jnp.float32),
            pltpu.VMEM_SHARED((npad if stage else 8, f), jnp.float32),
            pltpu.VMEM((nch // kblk, kblk * CH), jnp.int32),
            pltpu.VMEM((nch // kblk, kblk * CH), jnp.int32),
            pltpu.VMEM((nbuf, kblk * CH, f), jnp.float32),
            pltpu.VMEM((npad // NS, f), jnp.float32),
            [pltpu.SemaphoreType.DMA] * nbuf,
            [pltpu.SemaphoreType.DMA] * nbuf,
        ],
        compiler_params=pltpu.CompilerParams(use_tc_tiling_on_sc=False),
    )
    def prop_kernel(h_hbm, src_hbm, dst_hbm, out_hbm,
                    acc_sh, h_sh, sidx, didx, rows, bounce, gsem, ssem):
        c = lax.axis_index("c")
        s = lax.axis_index("s")
        wid = s * NC + c
        stripe = npad // NS
        sl = pl.ds(s * stripe, stripe)
        pltpu.sync_copy(h_hbm.at[sl], bounce)
        pltpu.sync_copy(bounce, acc_sh.at[sl])
        if stage:
            pltpu.sync_copy(bounce, h_sh.at[sl])
        pltpu.sync_copy(src_hbm.at[wid], sidx)
        pltpu.sync_copy(dst_hbm.at[wid], didx)
        plsc.subcore_barrier()
        gsrc = h_sh if stage else h_hbm

        def gather(q, b):
            pltpu.async_copy(gsrc.at[sidx.at[q]], rows.at[b], gsem[b])

        def gather_wait(q, b):
            pltpu.make_async_copy(gsrc.at[sidx.at[q]], rows.at[b], gsem[b]).wait()

        def scatter(q, b):
            pltpu.async_copy(rows.at[b], acc_sh.at[didx.at[q]], ssem[b], add=True)

        def scatter_wait(q, b):
            pltpu.make_async_copy(rows.at[b], acc_sh.at[didx.at[q]], ssem[b]).wait()

        for b in range(nbuf):
            gather(b, b)

        def body(g, carry):
            for b in range(nbuf):
                q = g * nbuf + b
                gather_wait(q, b)
                scatter(q, b)
            for b in range(nbuf):
                q = g * nbuf + b
                scatter_wait(q, b)

                @pl.when(g + 1 < ngrp)
                def _():
                    gather(q + nbuf, b)

            return carry

        lax.fori_loop(0, ngrp, body, 0)
        plsc.subcore_barrier()
        pltpu.sync_copy(acc_sh.at[sl], bounce)
        pltpu.sync_copy(bounce, out_hbm.at[c, sl])

    return prop_kernel


def _make_prop_cols(npad, nblk, fh, nbuf, chunk):
    nq = 2 * nblk
    assert nq % nbuf == 0
    ngrp = nq // nbuf

    @functools.partial(
        pl.kernel,
        mesh=_mesh(),
        out_type=jax.ShapeDtypeStruct((NC, npad, fh), jnp.float32),
        scratch_types=[
            pltpu.VMEM_SHARED((npad, fh), jnp.float32),
            pltpu.VMEM_SHARED((npad, fh), jnp.float32),
            pltpu.VMEM((nq, chunk), jnp.int32),
            pltpu.VMEM((nq, chunk), jnp.int32),
            pltpu.VMEM((nbuf, chunk, fh), jnp.float32),
            pltpu.VMEM((npad // NS, fh), jnp.float32),
            [pltpu.SemaphoreType.DMA] * nbuf,
            [pltpu.SemaphoreType.DMA] * nbuf,
        ],
        compiler_params=pltpu.CompilerParams(use_tc_tiling_on_sc=False),
    )
    def prop_kernel(hl_hbm, hr_hbm, src_hbm, dst_hbm, out_hbm,
                    acc_sh, h_sh, sidx, didx, rows, bounce, gsem, ssem):
        c = lax.axis_index("c")
        s = lax.axis_index("s")
        stripe = npad // NS
        sl = pl.ds(s * stripe, stripe)

        @pl.when(c == 0)
        def _():
            pltpu.sync_copy(hl_hbm.at[sl], bounce)

        @pl.when(c == 1)
        def _():
            pltpu.sync_copy(hr_hbm.at[sl], bounce)

        pltpu.sync_copy(bounce, acc_sh.at[sl])
        pltpu.sync_copy(bounce, h_sh.at[sl])
        pltpu.sync_copy(src_hbm.at[2 * s], sidx.at[pl.ds(0, nblk)])
        pltpu.sync_copy(src_hbm.at[2 * s + 1], sidx.at[pl.ds(nblk, nblk)])
        pltpu.sync_copy(dst_hbm.at[2 * s], didx.at[pl.ds(0, nblk)])
        pltpu.sync_copy(dst_hbm.at[2 * s + 1], didx.at[pl.ds(nblk, nblk)])
        plsc.subcore_barrier()

        def gather(q, b):
            pltpu.async_copy(h_sh.at[sidx.at[q]], rows.at[b], gsem[b])

        def gather_wait(q, b):
            pltpu.make_async_copy(h_sh.at[sidx.at[q]], rows.at[b], gsem[b]).wait()

        def scatter(q, b):
            pltpu.async_copy(rows.at[b], acc_sh.at[didx.at[q]], ssem[b], add=True)

        def scatter_wait(q, b):
            pltpu.make_async_copy(rows.at[b], acc_sh.at[didx.at[q]], ssem[b]).wait()

        for b in range(nbuf):
            gather(b, b)

        def body(g, carry):
            for b in range(nbuf):
                q = g * nbuf + b
                gather_wait(q, b)
                scatter(q, b)
            for b in range(nbuf):
                q = g * nbuf + b
                scatter_wait(q, b)

                @pl.when(g + 1 < ngrp)
                def _():
                    gather(q + nbuf, b)

            return carry

        lax.fori_loop(0, ngrp, body, 0)
        plsc.subcore_barrier()
        pltpu.sync_copy(acc_sh.at[sl], bounce)
        pltpu.sync_copy(bounce, out_hbm.at[c, sl])

    return prop_kernel


def _tc_first(x, w1, dinv, npad):
    n = x.shape[0]
    h1dim = w1.shape[1]
    fh = h1dim // 2

    def body(x_ref, w_ref, dinv_ref, hl_ref, hr_ref):
        h = jnp.dot(x_ref[...], w_ref[...], preferred_element_type=jnp.float32)
        hs = h * dinv_ref[...]
        rows = pl.ds(0, n)
        hl_ref[rows, :] = hs[:, :fh]
        hr_ref[rows, :] = hs[:, fh:]

    return pl.pallas_call(
        body,
        out_shape=(
            jax.ShapeDtypeStruct((npad, fh), jnp.float32),
            jax.ShapeDtypeStruct((npad, fh), jnp.float32),
        ),
    )(x, w1, dinv)


def _tc_mid(a0, a1, dinv, w2p):
    npad = a0.shape[0]
    n = dinv.shape[0]
    f2 = w2p.shape[1]

    def body(a0_ref, a1_ref, dinv_ref, w_ref, out_ref):
        rows = pl.ds(0, n)
        t = jnp.concatenate(
            [a0_ref[rows, :], a1_ref[rows, :]], axis=1) * dinv_ref[...]
        g = jnp.maximum(t, 0.0)
        h2 = jnp.dot(g, w_ref[...], preferred_element_type=jnp.float32)
        out_ref[rows, :] = h2 * dinv_ref[...]

    return pl.pallas_call(
        body,
        out_shape=jax.ShapeDtypeStruct((npad, f2), jnp.float32),
    )(a0, a1, dinv, w2p)


def _tc_final(a0, a1, h2s, dinv):
    f2 = a0.shape[1]
    n = dinv.shape[0]

    def body(a0_ref, a1_ref, h_ref, dinv_ref, out_ref):
        rows = pl.ds(0, n)
        out_ref[...] = (a0_ref[rows, :] + a1_ref[rows, :]
                        - h_ref[rows, :]) * dinv_ref[...]

    return pl.pallas_call(
        body,
        out_shape=jax.ShapeDtypeStruct((n, f2), jnp.float32),
    )(a0, a1, h2s, dinv)


def kernel(node_features, edge_features, latent_features, edge_index, device, W1, W2):
    x = node_features
    n = x.shape[0]
    e = edge_index.shape[1]
    h1dim = W1.shape[1]
    h2dim = W2.shape[1]
    f2 = 16

    kblk = 8
    chunk = kblk * CH
    nacc = ((n + 1 + 127) // 128) * 128
    ndeg = ((n + 1 + 255) // 256) * 256
    nblk = (e + NW * chunk - 1) // (NW * chunk)
    nblk = ((nblk + 1) // 2) * 2
    nch = nblk * kblk
    etot = NW * nblk * chunk

    src = edge_index[0]
    dst = edge_index[1]
    src_b = jnp.concatenate(
        [src, jnp.zeros((etot - e,), jnp.int32)]).reshape(NW, nblk, chunk)
    dst_b = jnp.concatenate(
        [dst, jnp.full((etot - e,), n, jnp.int32)]).reshape(NW, nblk, chunk)
    w2p = jnp.pad(W2, ((0, 0), (0, f2 - h2dim)))

    dinv = _make_dinv(ndeg, nblk, chunk)(dst_b)
    dinv2d = dinv.reshape(ndeg)[:n].reshape(n, 1)
    hl, hr = _tc_first(x, W1, dinv2d, nacc)
    acc1 = _make_prop_cols(nacc, nblk, h1dim // 2, 2, chunk)(hl, hr, src_b, dst_b)
    h2s = _tc_mid(acc1[0], acc1[1], dinv2d, w2p)
    acc2 = _make_prop(nacc, nch, f2, 2, kblk, True)(h2s, src_b, dst_b)
    outp = _tc_final(acc2[0], acc2[1], h2s, dinv2d)
    return outp[:, :h2dim]
